# baseline (speedup 1.0000x reference)
import functools
import jax
import jax.numpy as jnp
from jax import lax
from jax.experimental import pallas as pl
from jax.experimental.pallas import tpu as pltpu, tpu_sc as plsc

_NQ = 8192
_NB = 4
_ND = 2 * _NB

_TQ = 32
_CK = 128
_NC = _NQ // _CK

_S_SC = 18
_NQ_SC = _S_SC * 128
_NQ_TC = _NQ - _NQ_SC
_NT = _NQ_TC // _TQ


def _tc_body(q_ref, k_ref, min_ref, idx_ref, sum_ref):
    t = pl.program_id(1)
    q = q_ref[0]
    qxb = jnp.broadcast_to(q[:, 0:1], (_TQ, _CK))
    qyb = jnp.broadcast_to(q[:, 1:2], (_TQ, _CK))
    qzb = jnp.broadcast_to(q[:, 2:3], (_TQ, _CK))

    def step(j, carry):
        mmin, tid = carry
        k = k_ref[0, :, pl.ds(j * _CK, _CK)]
        dx = qxb - jnp.broadcast_to(k[0:1, :], (_TQ, _CK))
        dy = qyb - jnp.broadcast_to(k[1:2, :], (_TQ, _CK))
        dz = qzb - jnp.broadcast_to(k[2:3, :], (_TQ, _CK))
        d = dx * dx + dy * dy + dz * dz
        upd = d < mmin
        mmin = jnp.minimum(mmin, d)
        tid = jnp.where(upd, j.astype(jnp.float32), tid)
        return mmin, tid

    mmin = jnp.full((_TQ, _CK), jnp.inf, jnp.float32)
    tid = jnp.zeros((_TQ, _CK), jnp.float32)
    mmin, tid = jax.lax.fori_loop(0, _NC, step, (mmin, tid), unroll=8)

    m = jnp.min(mmin, axis=1)
    lane = jax.lax.broadcasted_iota(jnp.int32, (_TQ, _CK), 1).astype(jnp.float32)
    cand = tid * jnp.float32(_CK) + lane
    idxf = jnp.min(jnp.where(mmin == m[:, None], cand, jnp.float32(2 * _NQ)),
                   axis=1)
    min_ref[0, 0, :] = m
    idx_ref[0, 0, :] = idxf.astype(jnp.int32)

    @pl.when(t == 0)
    def _():
        sum_ref[0, 0, :] = jnp.zeros((_TQ,), jnp.float32)

    sum_ref[0, 0, :] += m


def _tc_nn(qs, ks):
    nd = qs.shape[0]
    grid = (nd, _NT)
    mins, idxs, sums = pl.pallas_call(
        _tc_body,
        grid=grid,
        in_specs=[
            pl.BlockSpec((1, _TQ, 3), lambda b, t: (b, t, 0)),
            pl.BlockSpec((1, 3, _NQ), lambda b, t: (b, 0, 0)),
        ],
        out_specs=[
            pl.BlockSpec((1, 1, _TQ), lambda b, t: (b * _NT + t, 0, 0)),
            pl.BlockSpec((1, 1, _TQ), lambda b, t: (b * _NT + t, 0, 0)),
            pl.BlockSpec((1, 1, _TQ), lambda b, t: (b, 0, 0)),
        ],
        out_shape=[
            jax.ShapeDtypeStruct((nd * _NT, 1, _TQ), jnp.float32),
            jax.ShapeDtypeStruct((nd * _NT, 1, _TQ), jnp.int32),
            jax.ShapeDtypeStruct((nd, 1, _TQ), jnp.float32),
        ],
    )(qs, ks)
    return idxs.reshape(nd, _NQ_TC), jnp.sum(sums)


_NW = 32
_QW = (_ND * _NQ_SC) // _NW
_QG = 4
_NG = _QW // (16 * _QG)


def _sc_body(qx_h, qy_h, qz_h, kx_h, ky_h, kz_h,
             idx_h, min_h, sum_h,
             qx_v, qy_v, qz_v, kx_v, ky_v, kz_v, min_v, idx_v, sum_v):
    wid = lax.axis_index("s") * 2 + lax.axis_index("c")
    qbase = wid * _QW
    kbase = (wid // 4) * _NQ

    pltpu.sync_copy(qx_h.at[pl.ds(qbase, _QW)], qx_v)
    pltpu.sync_copy(qy_h.at[pl.ds(qbase, _QW)], qy_v)
    pltpu.sync_copy(qz_h.at[pl.ds(qbase, _QW)], qz_v)
    pltpu.sync_copy(kx_h.at[pl.ds(kbase, _NQ)], kx_v)
    pltpu.sync_copy(ky_h.at[pl.ds(kbase, _NQ)], ky_v)
    pltpu.sync_copy(kz_h.at[pl.ds(kbase, _NQ)], kz_v)

    def group(g, ssum):
        qxs = [qx_v[pl.ds(g * 16 * _QG + 16 * r, 16)] for r in range(_QG)]
        qys = [qy_v[pl.ds(g * 16 * _QG + 16 * r, 16)] for r in range(_QG)]
        qzs = [qz_v[pl.ds(g * 16 * _QG + 16 * r, 16)] for r in range(_QG)]

        def step(kg, carry):
            ms = list(carry[:_QG])
            ts = list(carry[_QG:])
            kxg = kx_v[pl.ds(kg * 16, 16)]
            kyg = ky_v[pl.ds(kg * 16, 16)]
            kzg = kz_v[pl.ds(kg * 16, 16)]
            jbase = (kg * 16).astype(jnp.float32)
            for i in range(16):
                kxb = jnp.broadcast_to(kxg[i:i + 1], (16,))
                kyb = jnp.broadcast_to(kyg[i:i + 1], (16,))
                kzb = jnp.broadcast_to(kzg[i:i + 1], (16,))
                jf = jbase + jnp.float32(i)
                for r in range(_QG):
                    dx = qxs[r] - kxb
                    dy = qys[r] - kyb
                    dz = qzs[r] - kzb
                    d = dx * dx + dy * dy + dz * dz
                    u = d < ms[r]
                    ms[r] = jnp.minimum(ms[r], d)
                    ts[r] = jnp.where(u, jf, ts[r])
            return tuple(ms) + tuple(ts)

        inf = jnp.full((16,), jnp.inf, jnp.float32)
        zero = jnp.zeros((16,), jnp.float32)
        out = lax.fori_loop(0, _NQ // 16, step,
                            (inf,) * _QG + (zero,) * _QG)
        for r in range(_QG):
            min_v[pl.ds(g * 16 * _QG + 16 * r, 16)] = out[r]
            idx_v[pl.ds(g * 16 * _QG + 16 * r, 16)] = out[_QG + r].astype(jnp.int32)
            ssum = ssum + out[r]
        return ssum

    ssum = lax.fori_loop(0, _NG, group, jnp.zeros((16,), jnp.float32))
    sum_v[...] = ssum
    pltpu.sync_copy(min_v, min_h.at[pl.ds(qbase, _QW)])
    pltpu.sync_copy(idx_v, idx_h.at[pl.ds(qbase, _QW)])
    pltpu.sync_copy(sum_v, sum_h.at[pl.ds(wid * 16, 16)])


def _sc_nn(qx, qy, qz, kx, ky, kz):
    n = qx.shape[0]
    mesh = plsc.VectorSubcoreMesh(core_axis_name="c", subcore_axis_name="s")
    f = functools.partial(
        pl.kernel, mesh=mesh,
        out_type=[
            jax.ShapeDtypeStruct((n,), jnp.int32),
            jax.ShapeDtypeStruct((n,), jnp.float32),
            jax.ShapeDtypeStruct((_NW * 16,), jnp.float32),
        ],
        scratch_types=[
            pltpu.VMEM((_QW,), jnp.float32),
            pltpu.VMEM((_QW,), jnp.float32),
            pltpu.VMEM((_QW,), jnp.float32),
            pltpu.VMEM((_NQ,), jnp.float32),
            pltpu.VMEM((_NQ,), jnp.float32),
            pltpu.VMEM((_NQ,), jnp.float32),
            pltpu.VMEM((_QW,), jnp.float32),
            pltpu.VMEM((_QW,), jnp.int32),
            pltpu.VMEM((16,), jnp.float32),
        ],
    )(_sc_body)
    return f(qx, qy, qz, kx, ky, kz)


def kernel(pred_points, true_points):
    qs = jnp.concatenate([pred_points, true_points], axis=0)
    ks = jnp.concatenate([true_points, pred_points], axis=0)

    idx_tc, sum_tc = _tc_nn(qs[:, :_NQ_TC], ks.transpose(0, 2, 1))

    qsc = qs[:, _NQ_TC:]
    idx_sc, mins_sc, sums_sc = _sc_nn(
        qsc[..., 0].reshape(-1), qsc[..., 1].reshape(-1),
        qsc[..., 2].reshape(-1),
        ks[..., 0].reshape(-1), ks[..., 1].reshape(-1),
        ks[..., 2].reshape(-1),
    )

    loss = (sum_tc + jnp.sum(sums_sc)) / (_NB * _NQ)
    idxs = jnp.concatenate([idx_tc, idx_sc.reshape(_ND, _NQ_SC)], axis=1)
    return loss, idxs[:_NB], idxs[_NB:]


# speedup vs baseline: 1.0006x; 1.0006x over previous
import functools
import jax
import jax.numpy as jnp
from jax import lax
from jax.experimental import pallas as pl
from jax.experimental.pallas import tpu as pltpu, tpu_sc as plsc

_NQ = 8192
_NB = 4
_ND = 2 * _NB

_TQ = 32
_CK = 128
_NC = _NQ // _CK

_S_SC = 18
_NQ_SC = _S_SC * 128
_NQ_TC = _NQ - _NQ_SC
_NT = _NQ_TC // _TQ


def _tc_body(q_ref, k_ref, min_ref, idx_ref, sum_ref):
    t = pl.program_id(1)
    q = q_ref[0]
    qxb = jnp.broadcast_to(q[:, 0:1], (_TQ, _CK))
    qyb = jnp.broadcast_to(q[:, 1:2], (_TQ, _CK))
    qzb = jnp.broadcast_to(q[:, 2:3], (_TQ, _CK))

    def step(j, carry):
        mmin, tid = carry
        k = k_ref[0, :, pl.ds(j * _CK, _CK)]
        dx = qxb - jnp.broadcast_to(k[0:1, :], (_TQ, _CK))
        dy = qyb - jnp.broadcast_to(k[1:2, :], (_TQ, _CK))
        dz = qzb - jnp.broadcast_to(k[2:3, :], (_TQ, _CK))
        d = dx * dx + dy * dy + dz * dz
        upd = d < mmin
        mmin = jnp.minimum(mmin, d)
        tid = jnp.where(upd, j.astype(jnp.float32), tid)
        return mmin, tid

    mmin = jnp.full((_TQ, _CK), jnp.inf, jnp.float32)
    tid = jnp.zeros((_TQ, _CK), jnp.float32)
    mmin, tid = jax.lax.fori_loop(0, _NC, step, (mmin, tid), unroll=8)

    m = jnp.min(mmin, axis=1)
    lane = jax.lax.broadcasted_iota(jnp.int32, (_TQ, _CK), 1).astype(jnp.float32)
    cand = tid * jnp.float32(_CK) + lane
    idxf = jnp.min(jnp.where(mmin == m[:, None], cand, jnp.float32(2 * _NQ)),
                   axis=1)
    min_ref[0, 0, :] = m
    idx_ref[0, 0, :] = idxf.astype(jnp.int32)

    @pl.when(t == 0)
    def _():
        sum_ref[0, 0, :] = jnp.zeros((_TQ,), jnp.float32)

    sum_ref[0, 0, :] += m


def _tc_nn(qs, ks):
    nd = qs.shape[0]
    grid = (nd, _NT)
    mins, idxs, sums = pl.pallas_call(
        _tc_body,
        grid=grid,
        in_specs=[
            pl.BlockSpec((1, _TQ, 3), lambda b, t: (b, t, 0)),
            pl.BlockSpec((1, 3, _NQ), lambda b, t: (b, 0, 0)),
        ],
        out_specs=[
            pl.BlockSpec((1, 1, _TQ), lambda b, t: (b * _NT + t, 0, 0)),
            pl.BlockSpec((1, 1, _TQ), lambda b, t: (b * _NT + t, 0, 0)),
            pl.BlockSpec((1, 1, _TQ), lambda b, t: (b, 0, 0)),
        ],
        out_shape=[
            jax.ShapeDtypeStruct((nd * _NT, 1, _TQ), jnp.float32),
            jax.ShapeDtypeStruct((nd * _NT, 1, _TQ), jnp.int32),
            jax.ShapeDtypeStruct((nd, 1, _TQ), jnp.float32),
        ],
    )(qs, ks)
    return idxs.reshape(nd, _NQ_TC), jnp.sum(sums)


_NW = 32
_QW = (_ND * _NQ_SC) // _NW
_QG = 2
_NG = _QW // (16 * _QG)


def _sc_body(qx_h, qy_h, qz_h, kx_h, ky_h, kz_h,
             idx_h, min_h, sum_h,
             qx_v, qy_v, qz_v, kx_v, ky_v, kz_v, min_v, idx_v, sum_v):
    wid = lax.axis_index("s") * 2 + lax.axis_index("c")
    qbase = wid * _QW
    kbase = (wid // 4) * _NQ

    pltpu.sync_copy(qx_h.at[pl.ds(qbase, _QW)], qx_v)
    pltpu.sync_copy(qy_h.at[pl.ds(qbase, _QW)], qy_v)
    pltpu.sync_copy(qz_h.at[pl.ds(qbase, _QW)], qz_v)
    pltpu.sync_copy(kx_h.at[pl.ds(kbase, _NQ)], kx_v)
    pltpu.sync_copy(ky_h.at[pl.ds(kbase, _NQ)], ky_v)
    pltpu.sync_copy(kz_h.at[pl.ds(kbase, _NQ)], kz_v)

    def group(g, ssum):
        qxs = [qx_v[pl.ds(g * 16 * _QG + 16 * r, 16)] for r in range(_QG)]
        qys = [qy_v[pl.ds(g * 16 * _QG + 16 * r, 16)] for r in range(_QG)]
        qzs = [qz_v[pl.ds(g * 16 * _QG + 16 * r, 16)] for r in range(_QG)]

        def step(kg, carry):
            ms = list(carry[:_QG])
            ts = list(carry[_QG:])
            kxg = kx_v[pl.ds(kg * 16, 16)]
            kyg = ky_v[pl.ds(kg * 16, 16)]
            kzg = kz_v[pl.ds(kg * 16, 16)]
            jbase = (kg * 16).astype(jnp.float32)
            for i in range(16):
                kxb = jnp.broadcast_to(kxg[i:i + 1], (16,))
                kyb = jnp.broadcast_to(kyg[i:i + 1], (16,))
                kzb = jnp.broadcast_to(kzg[i:i + 1], (16,))
                jf = jbase + jnp.float32(i)
                for r in range(_QG):
                    dx = qxs[r] - kxb
                    dy = qys[r] - kyb
                    dz = qzs[r] - kzb
                    d = dx * dx + dy * dy + dz * dz
                    u = d < ms[r]
                    ms[r] = jnp.minimum(ms[r], d)
                    ts[r] = jnp.where(u, jf, ts[r])
            return tuple(ms) + tuple(ts)

        inf = jnp.full((16,), jnp.inf, jnp.float32)
        zero = jnp.zeros((16,), jnp.float32)
        out = lax.fori_loop(0, _NQ // 16, step,
                            (inf,) * _QG + (zero,) * _QG)
        for r in range(_QG):
            min_v[pl.ds(g * 16 * _QG + 16 * r, 16)] = out[r]
            idx_v[pl.ds(g * 16 * _QG + 16 * r, 16)] = out[_QG + r].astype(jnp.int32)
            ssum = ssum + out[r]
        return ssum

    ssum = lax.fori_loop(0, _NG, group, jnp.zeros((16,), jnp.float32))
    sum_v[...] = ssum
    pltpu.sync_copy(min_v, min_h.at[pl.ds(qbase, _QW)])
    pltpu.sync_copy(idx_v, idx_h.at[pl.ds(qbase, _QW)])
    pltpu.sync_copy(sum_v, sum_h.at[pl.ds(wid * 16, 16)])


def _sc_nn(qx, qy, qz, kx, ky, kz):
    n = qx.shape[0]
    mesh = plsc.VectorSubcoreMesh(core_axis_name="c", subcore_axis_name="s")
    f = functools.partial(
        pl.kernel, mesh=mesh,
        out_type=[
            jax.ShapeDtypeStruct((n,), jnp.int32),
            jax.ShapeDtypeStruct((n,), jnp.float32),
            jax.ShapeDtypeStruct((_NW * 16,), jnp.float32),
        ],
        scratch_types=[
            pltpu.VMEM((_QW,), jnp.float32),
            pltpu.VMEM((_QW,), jnp.float32),
            pltpu.VMEM((_QW,), jnp.float32),
            pltpu.VMEM((_NQ,), jnp.float32),
            pltpu.VMEM((_NQ,), jnp.float32),
            pltpu.VMEM((_NQ,), jnp.float32),
            pltpu.VMEM((_QW,), jnp.float32),
            pltpu.VMEM((_QW,), jnp.int32),
            pltpu.VMEM((16,), jnp.float32),
        ],
    )(_sc_body)
    return f(qx, qy, qz, kx, ky, kz)


def kernel(pred_points, true_points):
    qs = jnp.concatenate([pred_points, true_points], axis=0)
    ks = jnp.concatenate([true_points, pred_points], axis=0)

    idx_tc, sum_tc = _tc_nn(qs[:, :_NQ_TC], ks.transpose(0, 2, 1))

    qsc = qs[:, _NQ_TC:]
    idx_sc, mins_sc, sums_sc = _sc_nn(
        qsc[..., 0].reshape(-1), qsc[..., 1].reshape(-1),
        qsc[..., 2].reshape(-1),
        ks[..., 0].reshape(-1), ks[..., 1].reshape(-1),
        ks[..., 2].reshape(-1),
    )

    loss = (sum_tc + jnp.sum(sums_sc)) / (_NB * _NQ)
    idxs = jnp.concatenate([idx_tc, idx_sc.reshape(_ND, _NQ_SC)], axis=1)
    return loss, idxs[:_NB], idxs[_NB:]


# speedup vs baseline: 1.2972x; 1.2964x over previous
import functools
import jax
import jax.numpy as jnp
from jax import lax
from jax.experimental import pallas as pl
from jax.experimental.pallas import tpu as pltpu, tpu_sc as plsc

_NQ = 8192
_NB = 4
_ND = 2 * _NB

_TQ = 64
_CK = 128
_NC = _NQ // _CK

_S_SC = 18
_NQ_SC = _S_SC * 128
_NQ_TC = _NQ - _NQ_SC
_NT = _NQ_TC // _TQ


def _tc_body(q_ref, k_ref, min_ref, idx_ref, sum_ref):
    t = pl.program_id(1)
    q = q_ref[0]
    qxb = jnp.broadcast_to(q[:, 0:1], (_TQ, _CK))
    qyb = jnp.broadcast_to(q[:, 1:2], (_TQ, _CK))
    qzb = jnp.broadcast_to(q[:, 2:3], (_TQ, _CK))

    def step(j, carry):
        mmin, tid = carry
        k = k_ref[0, :, pl.ds(j * _CK, _CK)]
        dx = qxb - jnp.broadcast_to(k[0:1, :], (_TQ, _CK))
        dy = qyb - jnp.broadcast_to(k[1:2, :], (_TQ, _CK))
        dz = qzb - jnp.broadcast_to(k[2:3, :], (_TQ, _CK))
        d = dx * dx + dy * dy + dz * dz
        upd = d < mmin
        mmin = jnp.minimum(mmin, d)
        tid = jnp.where(upd, j.astype(jnp.float32), tid)
        return mmin, tid

    mmin = jnp.full((_TQ, _CK), jnp.inf, jnp.float32)
    tid = jnp.zeros((_TQ, _CK), jnp.float32)
    mmin, tid = jax.lax.fori_loop(0, _NC, step, (mmin, tid), unroll=8)

    m = jnp.min(mmin, axis=1)
    lane = jax.lax.broadcasted_iota(jnp.int32, (_TQ, _CK), 1).astype(jnp.float32)
    cand = tid * jnp.float32(_CK) + lane
    idxf = jnp.min(jnp.where(mmin == m[:, None], cand, jnp.float32(2 * _NQ)),
                   axis=1)
    min_ref[0, 0, :] = m
    idx_ref[0, 0, :] = idxf.astype(jnp.int32)

    @pl.when(t == 0)
    def _():
        sum_ref[0, 0, :] = jnp.zeros((_TQ,), jnp.float32)

    sum_ref[0, 0, :] += m


def _tc_nn(qs, ks):
    nd = qs.shape[0]
    grid = (nd, _NT)
    mins, idxs, sums = pl.pallas_call(
        _tc_body,
        grid=grid,
        in_specs=[
            pl.BlockSpec((1, _TQ, 3), lambda b, t: (b, t, 0)),
            pl.BlockSpec((1, 3, _NQ), lambda b, t: (b, 0, 0)),
        ],
        out_specs=[
            pl.BlockSpec((1, 1, _TQ), lambda b, t: (b * _NT + t, 0, 0)),
            pl.BlockSpec((1, 1, _TQ), lambda b, t: (b * _NT + t, 0, 0)),
            pl.BlockSpec((1, 1, _TQ), lambda b, t: (b, 0, 0)),
        ],
        out_shape=[
            jax.ShapeDtypeStruct((nd * _NT, 1, _TQ), jnp.float32),
            jax.ShapeDtypeStruct((nd * _NT, 1, _TQ), jnp.int32),
            jax.ShapeDtypeStruct((nd, 1, _TQ), jnp.float32),
        ],
    )(qs, ks)
    return idxs.reshape(nd, _NQ_TC), jnp.sum(sums)


_NW = 32
_QW = (_ND * _NQ_SC) // _NW
_QG = 2
_NG = _QW // (16 * _QG)


def _sc_body(qx_h, qy_h, qz_h, kx_h, ky_h, kz_h,
             idx_h, min_h, sum_h,
             qx_v, qy_v, qz_v, kx_v, ky_v, kz_v, min_v, idx_v, sum_v):
    wid = lax.axis_index("s") * 2 + lax.axis_index("c")
    qbase = wid * _QW
    kbase = (wid // 4) * _NQ

    pltpu.sync_copy(qx_h.at[pl.ds(qbase, _QW)], qx_v)
    pltpu.sync_copy(qy_h.at[pl.ds(qbase, _QW)], qy_v)
    pltpu.sync_copy(qz_h.at[pl.ds(qbase, _QW)], qz_v)
    pltpu.sync_copy(kx_h.at[pl.ds(kbase, _NQ)], kx_v)
    pltpu.sync_copy(ky_h.at[pl.ds(kbase, _NQ)], ky_v)
    pltpu.sync_copy(kz_h.at[pl.ds(kbase, _NQ)], kz_v)

    def group(g, ssum):
        qxs = [qx_v[pl.ds(g * 16 * _QG + 16 * r, 16)] for r in range(_QG)]
        qys = [qy_v[pl.ds(g * 16 * _QG + 16 * r, 16)] for r in range(_QG)]
        qzs = [qz_v[pl.ds(g * 16 * _QG + 16 * r, 16)] for r in range(_QG)]

        def step(kg, carry):
            ms = list(carry[:_QG])
            ts = list(carry[_QG:])
            kxg = kx_v[pl.ds(kg * 16, 16)]
            kyg = ky_v[pl.ds(kg * 16, 16)]
            kzg = kz_v[pl.ds(kg * 16, 16)]
            jbase = (kg * 16).astype(jnp.float32)
            for i in range(16):
                kxb = jnp.broadcast_to(kxg[i:i + 1], (16,))
                kyb = jnp.broadcast_to(kyg[i:i + 1], (16,))
                kzb = jnp.broadcast_to(kzg[i:i + 1], (16,))
                jf = jbase + jnp.float32(i)
                for r in range(_QG):
                    dx = qxs[r] - kxb
                    dy = qys[r] - kyb
                    dz = qzs[r] - kzb
                    d = dx * dx + dy * dy + dz * dz
                    u = d < ms[r]
                    ms[r] = jnp.minimum(ms[r], d)
                    ts[r] = jnp.where(u, jf, ts[r])
            return tuple(ms) + tuple(ts)

        inf = jnp.full((16,), jnp.inf, jnp.float32)
        zero = jnp.zeros((16,), jnp.float32)
        out = lax.fori_loop(0, _NQ // 16, step,
                            (inf,) * _QG + (zero,) * _QG)
        for r in range(_QG):
            min_v[pl.ds(g * 16 * _QG + 16 * r, 16)] = out[r]
            idx_v[pl.ds(g * 16 * _QG + 16 * r, 16)] = out[_QG + r].astype(jnp.int32)
            ssum = ssum + out[r]
        return ssum

    ssum = lax.fori_loop(0, _NG, group, jnp.zeros((16,), jnp.float32))
    sum_v[...] = ssum
    pltpu.sync_copy(min_v, min_h.at[pl.ds(qbase, _QW)])
    pltpu.sync_copy(idx_v, idx_h.at[pl.ds(qbase, _QW)])
    pltpu.sync_copy(sum_v, sum_h.at[pl.ds(wid * 16, 16)])


def _sc_nn(qx, qy, qz, kx, ky, kz):
    n = qx.shape[0]
    mesh = plsc.VectorSubcoreMesh(core_axis_name="c", subcore_axis_name="s")
    f = functools.partial(
        pl.kernel, mesh=mesh,
        out_type=[
            jax.ShapeDtypeStruct((n,), jnp.int32),
            jax.ShapeDtypeStruct((n,), jnp.float32),
            jax.ShapeDtypeStruct((_NW * 16,), jnp.float32),
        ],
        scratch_types=[
            pltpu.VMEM((_QW,), jnp.float32),
            pltpu.VMEM((_QW,), jnp.float32),
            pltpu.VMEM((_QW,), jnp.float32),
            pltpu.VMEM((_NQ,), jnp.float32),
            pltpu.VMEM((_NQ,), jnp.float32),
            pltpu.VMEM((_NQ,), jnp.float32),
            pltpu.VMEM((_QW,), jnp.float32),
            pltpu.VMEM((_QW,), jnp.int32),
            pltpu.VMEM((16,), jnp.float32),
        ],
    )(_sc_body)
    return f(qx, qy, qz, kx, ky, kz)


def kernel(pred_points, true_points):
    qs = jnp.concatenate([pred_points, true_points], axis=0)
    ks = jnp.concatenate([true_points, pred_points], axis=0)

    idx_tc, sum_tc = _tc_nn(qs[:, :_NQ_TC], ks.transpose(0, 2, 1))

    qsc = qs[:, _NQ_TC:]
    idx_sc, mins_sc, sums_sc = _sc_nn(
        qsc[..., 0].reshape(-1), qsc[..., 1].reshape(-1),
        qsc[..., 2].reshape(-1),
        ks[..., 0].reshape(-1), ks[..., 1].reshape(-1),
        ks[..., 2].reshape(-1),
    )

    loss = (sum_tc + jnp.sum(sums_sc)) / (_NB * _NQ)
    idxs = jnp.concatenate([idx_tc, idx_sc.reshape(_ND, _NQ_SC)], axis=1)
    return loss, idxs[:_NB], idxs[_NB:]
